# baseline (speedup 1.0000x reference)
import jax
import jax.numpy as jnp
from jax.experimental import pallas as pl
from jax.experimental.pallas import tpu as pltpu

_TN = 256


def _vq_block(x_ref, x2_ref, cb_ref, c2_ref, oh_ref, xhat_ref, idx_ref):
    G = cb_ref.shape[0]
    CB = cb_ref.shape[1]
    TN = x_ref.shape[0]
    for g in range(G):
        xg = x_ref[:, g, :]
        cbg = cb_ref[g, :, :]
        prod = jax.lax.dot_general(
            xg, cbg, (((1,), (1,)), ((), ())),
            preferred_element_type=jnp.float32)
        bias = x2_ref[:, g:g + 1] + c2_ref[g:g + 1, :]
        dist = bias - 2.0 * prod
        idx = jnp.argmin(dist, axis=1)
        iota = jax.lax.broadcasted_iota(jnp.int32, (TN, CB), 1)
        oh = (iota == idx[:, None]).astype(jnp.float32)
        oh_ref[:, g, :] = oh
        xhat_ref[:, g, :] = jnp.dot(
            oh, cbg, preferred_element_type=jnp.float32)
        idx_ref[:, g, :] = idx[:, None]


def kernel(x, code_book):
    n, G, dim = x.shape
    CB = code_book.shape[1]
    x2 = jnp.sum(x * x, axis=-1)
    c2 = jnp.sum(code_book * code_book, axis=-1)
    one_hot, x_hat, index = pl.pallas_call(
        _vq_block,
        grid=(n // _TN,),
        in_specs=[
            pl.BlockSpec((_TN, G, dim), lambda i: (i, 0, 0)),
            pl.BlockSpec((_TN, G), lambda i: (i, 0)),
            pl.BlockSpec((G, CB, dim), lambda i: (0, 0, 0)),
            pl.BlockSpec((G, CB), lambda i: (0, 0)),
        ],
        out_specs=[
            pl.BlockSpec((_TN, G, CB), lambda i: (i, 0, 0)),
            pl.BlockSpec((_TN, G, dim), lambda i: (i, 0, 0)),
            pl.BlockSpec((_TN, G, 1), lambda i: (i, 0, 0)),
        ],
        out_shape=[
            jax.ShapeDtypeStruct((n, G, CB), jnp.float32),
            jax.ShapeDtypeStruct((n, G, dim), jnp.float32),
            jax.ShapeDtypeStruct((n, G, 1), jnp.int32),
        ],
        compiler_params=pltpu.CompilerParams(
            dimension_semantics=("parallel",)),
    )(x, x2, code_book, c2)
    return (x_hat, one_hot, index)
